# baseline (speedup 1.0000x reference)
import jax
import jax.numpy as jnp
from jax.experimental import pallas as pl
from jax.experimental.pallas import tpu as pltpu

_THRESHOLDS = (0.5, 0.5, 0.5)
_NREL = 3


def _body(s_ref, n_ref, w_ref, o_ref, acc_ref):
    r = pl.program_id(1)
    t = jnp.where(
        r == 0,
        jnp.float32(_THRESHOLDS[0]),
        jnp.where(r == 1, jnp.float32(_THRESHOLDS[1]), jnp.float32(_THRESHOLDS[2])),
    )
    x = t * n_ref[0]

    @pl.when(r == 0)
    def _():
        acc_ref[...] = x

    @pl.when((r == 1) | (r == 2))
    def _():
        acc_ref[...] = acc_ref[...] + x

    @pl.when(r == _NREL)
    def _():
        agg = acc_ref[...] + s_ref[...]
        o_ref[...] = jnp.maximum(
            jnp.dot(agg, w_ref[...], preferred_element_type=jnp.float32), 0.0
        )


def kernel(self_feats, neigh_feats, weight):
    n, f = self_feats.shape
    e = weight.shape[1]
    block = 4000
    assert n % block == 0
    neigh3 = neigh_feats.reshape(_NREL, n, f)
    return pl.pallas_call(
        _body,
        grid=(n // block, _NREL + 1),
        in_specs=[
            pl.BlockSpec((block, f), lambda i, r: (i, 0)),
            pl.BlockSpec((1, block, f), lambda i, r: (jnp.minimum(r, 2), i, 0)),
            pl.BlockSpec((f, e), lambda i, r: (0, 0)),
        ],
        out_specs=pl.BlockSpec((block, e), lambda i, r: (i, 0)),
        out_shape=jax.ShapeDtypeStruct((n, e), jnp.float32),
        scratch_shapes=[pltpu.VMEM((block, f), jnp.float32)],
        compiler_params=pltpu.CompilerParams(
            dimension_semantics=("arbitrary", "arbitrary"),
        ),
    )(self_feats, neigh3, weight)


# speedup vs baseline: 1.5883x; 1.5883x over previous
import jax
import jax.numpy as jnp
from jax.experimental import pallas as pl
from jax.experimental.pallas import tpu as pltpu

_THRESHOLDS = (0.5, 0.5, 0.5)


def _body(s_ref, n_ref, w_ref, o_ref):
    agg = s_ref[...]
    for r, t in enumerate(_THRESHOLDS):
        agg = agg + t * n_ref[r]
    o_ref[...] = jnp.maximum(
        jnp.dot(agg, w_ref[...], preferred_element_type=jnp.float32), 0.0
    )


def kernel(self_feats, neigh_feats, weight):
    n, f = self_feats.shape
    e = weight.shape[1]
    nrel = neigh_feats.shape[0] // n
    block = 4000
    assert n % block == 0
    neigh3 = neigh_feats.reshape(nrel, n, f)
    return pl.pallas_call(
        _body,
        grid=(n // block,),
        in_specs=[
            pl.BlockSpec((block, f), lambda i: (i, 0)),
            pl.BlockSpec((nrel, block, f), lambda i: (0, i, 0)),
            pl.BlockSpec((f, e), lambda i: (0, 0)),
        ],
        out_specs=pl.BlockSpec((block, e), lambda i: (i, 0)),
        out_shape=jax.ShapeDtypeStruct((n, e), jnp.float32),
        compiler_params=pltpu.CompilerParams(
            dimension_semantics=("arbitrary",),
        ),
    )(self_feats, neigh3, weight)


# speedup vs baseline: 1.5890x; 1.0004x over previous
import jax
import jax.numpy as jnp
from jax.experimental import pallas as pl
from jax.experimental.pallas import tpu as pltpu

_THRESHOLDS = (0.5, 0.5, 0.5)


def _body(s_ref, n_ref, w_ref, o_ref):
    agg = s_ref[...]
    for r, t in enumerate(_THRESHOLDS):
        agg = agg + t * n_ref[r]
    o_ref[...] = jnp.maximum(
        jnp.dot(agg, w_ref[...], preferred_element_type=jnp.float32), 0.0
    )


def kernel(self_feats, neigh_feats, weight):
    n, f = self_feats.shape
    e = weight.shape[1]
    nrel = neigh_feats.shape[0] // n
    block = 4000
    assert n % block == 0
    neigh3 = neigh_feats.reshape(nrel, n, f)
    return pl.pallas_call(
        _body,
        grid=(n // block,),
        in_specs=[
            pl.BlockSpec((block, f), lambda i: (i, 0)),
            pl.BlockSpec((nrel, block, f), lambda i: (0, i, 0)),
            pl.BlockSpec((f, e), lambda i: (0, 0)),
        ],
        out_specs=pl.BlockSpec((block, e), lambda i: (i, 0)),
        out_shape=jax.ShapeDtypeStruct((n, e), jnp.float32),
        compiler_params=pltpu.CompilerParams(
            dimension_semantics=("parallel",),
        ),
    )(self_feats, neigh3, weight)
